# Initial kernel scaffold; baseline (speedup 1.0000x reference)
import jax
import jax.numpy as jnp
from jax.experimental import pallas as pl


def kernel(word, pos1, pos2, word_table, pos1_table, pos2_table):
    raise NotImplementedError("write your pallas kernel here")


# speedup vs baseline: 7.3510x; 7.3510x over previous
import functools

import jax
import jax.numpy as jnp
from jax import lax
from jax.experimental import pallas as pl
from jax.experimental.pallas import tpu as pltpu
from jax.experimental.pallas import tpu_sc as plsc

B = 1024
L = 200
N = B * L
WORD_DIM = 128
POS_SIZE = 16
OUT_DIM = WORD_DIM + 2 * POS_SIZE

NC, NS = 2, 16
NW = NC * NS
N_PER_W = N // NW
CH = 128
NCH = N_PER_W // CH
NG = CH // 16


def _make_kernel():
    mesh = plsc.VectorSubcoreMesh(core_axis_name="c", subcore_axis_name="s")

    @functools.partial(
        pl.kernel,
        mesh=mesh,
        out_type=jax.ShapeDtypeStruct((N, OUT_DIM), jnp.float32),
        compiler_params=pltpu.CompilerParams(needs_layout_passes=False),
        scratch_types=[
            pltpu.VMEM((NCH, CH), jnp.int32),
            pltpu.VMEM((NCH, CH), jnp.int32),
            pltpu.VMEM((NCH, CH), jnp.int32),
            pltpu.VMEM((64, 128), jnp.float32),
            pltpu.VMEM((64, 128), jnp.float32),
            pltpu.VMEM((CH, OUT_DIM), jnp.float32),
            pltpu.SemaphoreType.DMA,
        ],
    )
    def lookup(word_i, pos1_i, pos2_i, wtab, p1tab, p2tab, out,
               widx_v, p1idx_v, p2idx_v, p1tab_v, p2tab_v, obuf, sem):
        wid = lax.axis_index("s") * NC + lax.axis_index("c")
        pltpu.sync_copy(word_i.at[wid], widx_v)
        pltpu.sync_copy(pos1_i.at[wid], p1idx_v)
        pltpu.sync_copy(pos2_i.at[wid], p2idx_v)
        pltpu.sync_copy(p1tab, p1tab_v)
        pltpu.sync_copy(p2tab, p2tab_v)
        base = wid * N_PER_W
        lanes = lax.iota(jnp.int32, 16)

        def place16(idx_v, tab_v, j, g, col0):
            rr = idx_v[j, pl.ds(g * 16, 16)]
            prow = lax.shift_right_logical(rr, 3)
            pcol = lax.shift_left(lax.bitwise_and(rr, 7), 4)
            orow = g * 16 + lanes
            for kk in range(16):
                c = lax.bitwise_and(lanes + kk, 15)
                vals = plsc.load_gather(tab_v, [prow, pcol + c])
                plsc.store_scatter(obuf, [orow, c + col0], vals)

        def chunk(j, carry):
            cw = pltpu.async_copy(wtab.at[widx_v.at[j]],
                                  obuf.at[:, pl.ds(0, WORD_DIM)], sem)

            def group(g, c):
                place16(p1idx_v, p1tab_v, j, g, WORD_DIM)
                place16(p2idx_v, p2tab_v, j, g, WORD_DIM + POS_SIZE)
                return c

            lax.fori_loop(0, NG, group, 0)
            cw.wait()
            pltpu.sync_copy(obuf, out.at[pl.ds(base + j * CH, CH)])
            return carry

        lax.fori_loop(0, NCH, chunk, 0)

    return lookup


_LOOKUP = _make_kernel()


def kernel(word, pos1, pos2, word_table, pos1_table, pos2_table):
    wf = word.reshape(NW, NCH, CH).astype(jnp.int32)
    p1f = pos1.reshape(NW, NCH, CH).astype(jnp.int32)
    p2f = pos2.reshape(NW, NCH, CH).astype(jnp.int32)
    p1t = pos1_table.reshape(64, 128)
    p2t = pos2_table.reshape(64, 128)
    out = _LOOKUP(wf, p1f, p2f, word_table, p1t, p2t)
    return out.reshape(B, L, OUT_DIM)


# speedup vs baseline: 8.0259x; 1.0918x over previous
import functools

import jax
import jax.numpy as jnp
from jax import lax
from jax.experimental import pallas as pl
from jax.experimental.pallas import tpu as pltpu
from jax.experimental.pallas import tpu_sc as plsc

B = 1024
L = 200
N = B * L
WORD_DIM = 128
POS_SIZE = 16
OUT_DIM = WORD_DIM + 2 * POS_SIZE

NC, NS = 2, 16
NW = NC * NS
ROWS_PER_W = B // NW
N_PER_W = N // NW
CH = 128
NCH = N_PER_W // CH
NG = CH // 16


def _make_kernel():
    mesh = plsc.VectorSubcoreMesh(core_axis_name="c", subcore_axis_name="s")

    @functools.partial(
        pl.kernel,
        mesh=mesh,
        out_type=jax.ShapeDtypeStruct((N, OUT_DIM), jnp.float32),
        compiler_params=pltpu.CompilerParams(needs_layout_passes=False),
        scratch_types=[
            pltpu.VMEM((ROWS_PER_W, L), jnp.int32),
            pltpu.VMEM((NCH, CH), jnp.int32),
            pltpu.VMEM((NCH, CH), jnp.int32),
            pltpu.VMEM((NCH, CH), jnp.int32),
            pltpu.VMEM((64, 128), jnp.float32),
            pltpu.VMEM((64, 128), jnp.float32),
            pltpu.VMEM((2 * CH, OUT_DIM), jnp.float32),
            pltpu.SemaphoreType.DMA,
            pltpu.SemaphoreType.DMA,
        ],
    )
    def lookup(word_i, pos1_i, pos2_i, wtab, p1tab, p2tab, out,
               nat_v, widx_v, p1idx_v, p2idx_v, p1tab_v, p2tab_v, obuf,
               gsem, wsem):
        wid = lax.axis_index("s") * NC + lax.axis_index("c")
        lanes = lax.iota(jnp.int32, 16)
        l200 = jnp.full((16,), L, jnp.int32)

        def repack(src_h, dst_v):
            pltpu.sync_copy(src_h.at[pl.ds(wid * ROWS_PER_W, ROWS_PER_W)],
                            nat_v)

            def rp(t, c):
                nv = jnp.full((16,), t * 16, jnp.int32) + lanes
                rowv = lax.div(nv, l200)
                colv = nv - rowv * l200
                vals = plsc.load_gather(nat_v, [rowv, colv])
                r = jnp.full((16,), lax.shift_right_logical(t, 3), jnp.int32)
                c0 = lax.shift_left(lax.bitwise_and(t, 7), 4)
                plsc.store_scatter(
                    dst_v, [r, jnp.full((16,), c0, jnp.int32) + lanes], vals)
                return c

            lax.fori_loop(0, N_PER_W // 16, rp, 0)

        repack(word_i, widx_v)
        repack(pos1_i, p1idx_v)
        repack(pos2_i, p2idx_v)
        pltpu.sync_copy(p1tab, p1tab_v)
        pltpu.sync_copy(p2tab, p2tab_v)
        base = wid * N_PER_W

        def fire_gather(j, sbase):
            return pltpu.async_copy(
                wtab.at[widx_v.at[j]],
                obuf.at[pl.ds(sbase, CH), pl.ds(0, WORD_DIM)], gsem)

        def place_pos(j, sbase):
            for tab_v, idx_v, col0 in ((p1tab_v, p1idx_v, WORD_DIM),
                                       (p2tab_v, p2idx_v, WORD_DIM + POS_SIZE)):
                for g in range(NG):
                    rr = idx_v[j, pl.ds(g * 16, 16)]
                    prow = lax.shift_right_logical(rr, 3)
                    pcol = lax.shift_left(lax.bitwise_and(rr, 7), 4)
                    orow = jnp.full((16,), sbase + g * 16, jnp.int32) + lanes
                    for kk in range(16):
                        c = lax.bitwise_and(lanes + kk, 15)
                        vals = plsc.load_gather(tab_v, [prow, pcol + c])
                        plsc.store_scatter(obuf, [orow, c + col0], vals)

        fire_gather(0, 0)

        def chunk(j, carry):
            s = lax.bitwise_and(j, 1)
            sbase = s * CH
            nsbase = CH - sbase
            place_pos(j, sbase)
            pltpu.make_async_copy(
                wtab.at[widx_v.at[j]],
                obuf.at[pl.ds(sbase, CH), pl.ds(0, WORD_DIM)], gsem).wait()
            pltpu.async_copy(obuf.at[pl.ds(sbase, CH)],
                             out.at[pl.ds(base + j * CH, CH)], wsem)

            @pl.when(j > 0)
            def _():
                pltpu.make_async_copy(
                    out.at[pl.ds(base, CH)],
                    obuf.at[pl.ds(nsbase, CH)], wsem).wait()

            @pl.when(j < NCH - 1)
            def _():
                fire_gather(j + 1, nsbase)

            return carry

        lax.fori_loop(0, NCH, chunk, 0)
        pltpu.make_async_copy(out.at[pl.ds(base, CH)],
                              obuf.at[pl.ds(CH, CH)], wsem).wait()

    return lookup


_LOOKUP = _make_kernel()


def kernel(word, pos1, pos2, word_table, pos1_table, pos2_table):
    wf = word.astype(jnp.int32)
    p1f = pos1.astype(jnp.int32)
    p2f = pos2.astype(jnp.int32)
    p1t = pos1_table.reshape(64, 128)
    p2t = pos2_table.reshape(64, 128)
    out = _LOOKUP(wf, p1f, p2f, word_table, p1t, p2t)
    return out.reshape(B, L, OUT_DIM)
